# baseline (speedup 1.0000x reference)
import functools

import jax
import jax.numpy as jnp
from jax import lax
from jax.experimental import pallas as pl
from jax.experimental.pallas import tpu as pltpu
from jax.experimental.pallas import tpu_sc as plsc

_N, _S, _K, _C = 4096, 20, 10, 200
_NEG = -1000000000.0
_PADID = 204
_LN2 = 0.6931471805599453
_LOGC = (
    -0.024825606615738415,
    0.2668588228733106,
    -1.234263173084068,
    3.218832837151809,
    -5.264110477180785,
    6.065830143240842,
    -3.0283174810522713,
)

_NW = 32
_NB = _N // _NW
_HS = _S // 2


def _ln(sv):
    xi = lax.bitcast_convert_type(sv, jnp.int32)
    ee = ((xi >> 23) - 127).astype(jnp.float32)
    mm = lax.bitcast_convert_type((xi & 0x7FFFFF) | 0x3F800000, jnp.float32)
    p = jnp.full((16,), _LOGC[0], jnp.float32)
    for c in _LOGC[1:]:
        p = p * mm + c
    return (ee + p) * _LN2


def _sc_body(itemsT_hbm, banT_hbm, out_hbm, itemsT_v, banT_v, counts_v,
             e_v, prev_v, esub_v, lse_v, outbuf_v):
    wid = lax.axis_index("s") * 2 + lax.axis_index("c")
    base = wid * _NB

    pltpu.sync_copy(itemsT_hbm.at[:, pl.ds(base, _NB)], itemsT_v)
    pltpu.sync_copy(banT_hbm.at[:, :, pl.ds(base, _NB)], banT_v)

    iota16 = lax.iota(jnp.int32, 16)
    zeros_f = jnp.zeros((16,), jnp.float32)
    ones_f = jnp.ones((16,), jnp.float32)
    kaddr = iota16 * (_S * _NB)

    def _zero(i, _):
        counts_v[pl.ds(i * 16, 16)] = zeros_f
        return 0

    lax.fori_loop(0, _C * _NB // 16, _zero, 0)
    for c in range(_C, 208):
        e_v[pl.ds(c * 16, 16)] = zeros_f
    for s in range(_S):
        prev_v[pl.ds(24 * s, 16)] = jnp.full((16,), -1, jnp.int32)

    for g in range(_NB // 16):
        rows = iota16 + 16 * g
        last = jnp.zeros((16,), jnp.int32)
        for j in range(_S):
            idx = itemsT_v[j, pl.ds(16 * g, 16)]
            valid = idx != 0
            plsc.addupdate_scatter(counts_v, [idx * _NB + rows], ones_f,
                                   mask=valid)
            last = jnp.where(valid, idx, last)
        plsc.addupdate_scatter(counts_v, [last * _NB + rows], -ones_f,
                               mask=last != 0)

    lane_pad = iota16 >= _K
    negs = jnp.full((16,), _NEG, jnp.float32)

    def _group(g, _):
        go = pl.multiple_of(16 * g, 16)
        def _mx(c, m):
            return jnp.maximum(m, counts_v[pl.ds(c * _NB + go, 16)])

        mx = lax.fori_loop(1, _C, _mx, counts_v[pl.ds(go, 16)])

        def _et(c, acc):
            e = jnp.exp(counts_v[pl.ds(c * _NB + go, 16)] - mx)
            e_v[pl.ds(c * 16, 16)] = e
            return acc + e

        sacc = lax.fori_loop(0, _C, _et, zeros_f)

        def _chain(r, _):
            nl = go + r
            nlv = jnp.full((16,), nl, jnp.int32)
            for s in range(_S):
                braw = plsc.load_gather(
                    banT_v, [iota16, jnp.full((16,), s, jnp.int32), nlv]
                )
                b = jnp.where(lane_pad, _PADID, braw)
                sk = lax.sort(b)
                plsc.store_scatter(prev_v, [iota16 + 1 + 24 * s], sk)
                prev = prev_v[pl.ds(24 * s, 16)]
                eb = plsc.load_gather(e_v, [sk * 16 + r])
                esub = jnp.sum(jnp.where(sk != prev, eb, 0.0))
                plsc.store_scatter(
                    esub_v, [jnp.full((16,), s * 16, jnp.int32) + r],
                    jnp.full((16,), esub, jnp.float32),
                    mask=iota16 == 0,
                )
            return 0

        lax.fori_loop(0, 16, _chain, 0)

        for s in range(_S):
            sv = sacc - esub_v[pl.ds(s * 16, 16)]
            lse_v[pl.ds(s * 16, 16)] = _ln(sv) + mx

        for h in range(2):
            for sl in range(_HS):
                lse = lse_v[pl.ds((h * _HS + sl) * 16, 16)]

                def _orow(c, _, sl=sl, lse=lse):
                    outbuf_v[sl * _C + c, pl.ds(0, 16)] = (
                        counts_v[pl.ds(c * _NB + go, 16)] - lse
                    )
                    return 0

                lax.fori_loop(0, _C, _orow, 0)

            def _banscat(r, _, h=h):
                nlv = jnp.full((16,), go + r, jnp.int32)
                for sl in range(_HS):
                    s = h * _HS + sl
                    braw = plsc.load_gather(
                        banT_v, [iota16, jnp.full((16,), s, jnp.int32), nlv]
                    )
                    b = jnp.where(lane_pad, _PADID, braw)
                    plsc.store_scatter(
                        outbuf_v,
                        [jnp.full((16,), sl * _C, jnp.int32) + b,
                         jnp.full((16,), r, jnp.int32)],
                        negs, mask=b < _C,
                    )
                return 0

            lax.fori_loop(0, 16, _banscat, 0)
            pltpu.sync_copy(
                outbuf_v,
                out_hbm.at[wid, pl.ds(h * _HS * _C, _HS * _C),
                           pl.ds(go, 16)],
            )
        return 0

    lax.fori_loop(0, _NB // 16, _group, 0)


_ROWS = _S * _C


def _tc_unchunk(in_ref, out_ref):
    out_ref[...] = in_ref[0].reshape(_S, _C, 128)


def kernel(ban_ids, item_ids, aux1, aux2, aux3):
    del aux1, aux2, aux3
    itemsT = jnp.transpose(item_ids.astype(jnp.int32))
    banT = jnp.transpose(ban_ids.astype(jnp.int32), (2, 1, 0))

    mesh = plsc.VectorSubcoreMesh(core_axis_name="c", subcore_axis_name="s")
    run = functools.partial(
        pl.kernel,
        out_type=jax.ShapeDtypeStruct((_NW, _ROWS, _NB), jnp.float32),
        mesh=mesh,
        compiler_params=pltpu.CompilerParams(
            use_tc_tiling_on_sc=False, needs_layout_passes=False
        ),
        scratch_types=[
            pltpu.VMEM((_S, _NB), jnp.int32),
            pltpu.VMEM((_K, _S, _NB), jnp.int32),
            pltpu.VMEM((_C * _NB,), jnp.float32),
            pltpu.VMEM((208 * 16,), jnp.float32),
            pltpu.VMEM((24 * _S,), jnp.int32),
            pltpu.VMEM((16 * _S,), jnp.float32),
            pltpu.VMEM((16 * _S,), jnp.float32),
            pltpu.VMEM((_HS * _C, 16), jnp.float32),
        ],
    )(_sc_body)

    chunked = run(itemsT, banT)

    pi_t = pl.pallas_call(
        _tc_unchunk,
        grid=(_NW,),
        in_specs=[pl.BlockSpec((1, _ROWS, _NB), lambda i: (i, 0, 0))],
        out_specs=pl.BlockSpec((_S, _C, _NB), lambda i: (0, 0, i)),
        out_shape=jax.ShapeDtypeStruct((_S, _C, _N), jnp.float32),
        compiler_params=pltpu.CompilerParams(
            dimension_semantics=("parallel",),
        ),
    )(chunked)

    pi = jnp.transpose(pi_t, (2, 0, 1))
    v = jnp.zeros((_N, _S, 1), jnp.float32)
    return (pi, v)


# speedup vs baseline: 1.0736x; 1.0736x over previous
import functools

import jax
import jax.numpy as jnp
from jax import lax
from jax.experimental import pallas as pl
from jax.experimental.pallas import tpu as pltpu
from jax.experimental.pallas import tpu_sc as plsc

_N, _S, _K, _C = 4096, 20, 10, 200
_NEG = -1000000000.0
_PADID = 204
_LN2 = 0.6931471805599453
_LOGC = (
    -0.024825606615738415,
    0.2668588228733106,
    -1.234263173084068,
    3.218832837151809,
    -5.264110477180785,
    6.065830143240842,
    -3.0283174810522713,
)

_NW = 32
_NB = _N // _NW
_HS = _S // 2


def _ln(sv):
    xi = lax.bitcast_convert_type(sv, jnp.int32)
    ee = ((xi >> 23) - 127).astype(jnp.float32)
    mm = lax.bitcast_convert_type((xi & 0x7FFFFF) | 0x3F800000, jnp.float32)
    p = jnp.full((16,), _LOGC[0], jnp.float32)
    for c in _LOGC[1:]:
        p = p * mm + c
    return (ee + p) * _LN2


def _sc_body(itemsT_hbm, banT_hbm, out_hbm, itemsT_v, banT_v, counts_v,
             e_v, prev_v, esub_v, lse_v, outbuf_v, osem):
    wid = lax.axis_index("s") * 2 + lax.axis_index("c")
    base = wid * _NB

    pltpu.sync_copy(itemsT_hbm.at[:, pl.ds(base, _NB)], itemsT_v)
    pltpu.sync_copy(banT_hbm.at[:, :, pl.ds(base, _NB)], banT_v)

    iota16 = lax.iota(jnp.int32, 16)
    zeros_f = jnp.zeros((16,), jnp.float32)
    ones_f = jnp.ones((16,), jnp.float32)
    kaddr = iota16 * (_S * _NB)

    def _zero(i, _):
        counts_v[pl.ds(i * 16, 16)] = zeros_f
        return 0

    lax.fori_loop(0, _C * _NB // 16, _zero, 0)
    for c in range(_C, 208):
        e_v[pl.ds(c * 16, 16)] = zeros_f
    for s in range(_S):
        prev_v[pl.ds(24 * s, 16)] = jnp.full((16,), -1, jnp.int32)

    for g in range(_NB // 16):
        rows = iota16 + 16 * g
        last = jnp.zeros((16,), jnp.int32)
        for j in range(_S):
            idx = itemsT_v[j, pl.ds(16 * g, 16)]
            valid = idx != 0
            plsc.addupdate_scatter(counts_v, [idx * _NB + rows], ones_f,
                                   mask=valid)
            last = jnp.where(valid, idx, last)
        plsc.addupdate_scatter(counts_v, [last * _NB + rows], -ones_f,
                               mask=last != 0)

    lane_pad = iota16 >= _K
    negs = jnp.full((16,), _NEG, jnp.float32)

    def _group(g, _):
        go = pl.multiple_of(16 * g, 16)
        def _mx(c, m):
            return jnp.maximum(m, counts_v[pl.ds(c * _NB + go, 16)])

        mx = lax.fori_loop(1, _C, _mx, counts_v[pl.ds(go, 16)])

        def _et(c, acc):
            e = jnp.exp(counts_v[pl.ds(c * _NB + go, 16)] - mx)
            e_v[pl.ds(c * 16, 16)] = e
            return acc + e

        sacc = lax.fori_loop(0, _C, _et, zeros_f)

        def _chain(r, _):
            nl = go + r
            nlv = jnp.full((16,), nl, jnp.int32)
            for s in range(_S):
                braw = plsc.load_gather(
                    banT_v, [iota16, jnp.full((16,), s, jnp.int32), nlv]
                )
                b = jnp.where(lane_pad, _PADID, braw)
                sk = lax.sort(b)
                plsc.store_scatter(prev_v, [iota16 + 1 + 24 * s], sk)
                prev = prev_v[pl.ds(24 * s, 16)]
                eb = plsc.load_gather(e_v, [sk * 16 + r])
                esub = jnp.sum(jnp.where(sk != prev, eb, 0.0))
                plsc.store_scatter(
                    esub_v, [jnp.full((16,), s * 16, jnp.int32) + r],
                    jnp.full((16,), esub, jnp.float32),
                    mask=iota16 == 0,
                )
            return 0

        lax.fori_loop(0, 16, _chain, 0)

        for s in range(_S):
            sv = sacc - esub_v[pl.ds(s * 16, 16)]
            lse_v[pl.ds(s * 16, 16)] = _ln(sv) + mx

        for h in range(2):
            @pl.when(g >= 1)
            def _wait_prev(h=h):
                pltpu.make_async_copy(
                    outbuf_v.at[h],
                    out_hbm.at[wid, pl.ds(h * _HS * _C, _HS * _C),
                               pl.ds(go, 16)],
                    osem.at[h],
                ).wait()

            for sl in range(_HS):
                lse = lse_v[pl.ds((h * _HS + sl) * 16, 16)]

                def _orow(cb, _, sl=sl, lse=lse, h=h):
                    for dc in range(8):
                        outbuf_v[h, sl * _C + cb * 8 + dc, pl.ds(0, 16)] = (
                            counts_v[pl.ds(cb * 8 * _NB + dc * _NB + go, 16)]
                            - lse
                        )
                    return 0

                lax.fori_loop(0, _C // 8, _orow, 0)

            def _banscat(r, _, h=h):
                nlv = jnp.full((16,), go + r, jnp.int32)
                for sl in range(_HS):
                    s = h * _HS + sl
                    braw = plsc.load_gather(
                        banT_v, [iota16, jnp.full((16,), s, jnp.int32), nlv]
                    )
                    b = jnp.where(lane_pad, _PADID, braw)
                    plsc.store_scatter(
                        outbuf_v.at[h],
                        [jnp.full((16,), sl * _C, jnp.int32) + b,
                         jnp.full((16,), r, jnp.int32)],
                        negs, mask=b < _C,
                    )
                return 0

            lax.fori_loop(0, 16, _banscat, 0)
            pltpu.make_async_copy(
                outbuf_v.at[h],
                out_hbm.at[wid, pl.ds(h * _HS * _C, _HS * _C),
                           pl.ds(go, 16)],
                osem.at[h],
            ).start()
        return 0

    lax.fori_loop(0, _NB // 16, _group, 0)
    lastgo = _NB - 16
    for h in range(2):
        pltpu.make_async_copy(
            outbuf_v.at[h],
            out_hbm.at[wid, pl.ds(h * _HS * _C, _HS * _C),
                       pl.ds(lastgo, 16)],
            osem.at[h],
        ).wait()


_ROWS = _S * _C


def _tc_unchunk(in_ref, out_ref):
    out_ref[...] = in_ref[0].reshape(_S, _C, 128)


def kernel(ban_ids, item_ids, aux1, aux2, aux3):
    del aux1, aux2, aux3
    itemsT = jnp.transpose(item_ids.astype(jnp.int32))
    banT = jnp.transpose(ban_ids.astype(jnp.int32), (2, 1, 0))

    mesh = plsc.VectorSubcoreMesh(core_axis_name="c", subcore_axis_name="s")
    run = functools.partial(
        pl.kernel,
        out_type=jax.ShapeDtypeStruct((_NW, _ROWS, _NB), jnp.float32),
        mesh=mesh,
        compiler_params=pltpu.CompilerParams(
            use_tc_tiling_on_sc=False, needs_layout_passes=False
        ),
        scratch_types=[
            pltpu.VMEM((_S, _NB), jnp.int32),
            pltpu.VMEM((_K, _S, _NB), jnp.int32),
            pltpu.VMEM((_C * _NB,), jnp.float32),
            pltpu.VMEM((208 * 16,), jnp.float32),
            pltpu.VMEM((24 * _S,), jnp.int32),
            pltpu.VMEM((16 * _S,), jnp.float32),
            pltpu.VMEM((16 * _S,), jnp.float32),
            pltpu.VMEM((2, _HS * _C, 16), jnp.float32),
            pltpu.SemaphoreType.DMA((2,)),
        ],
    )(_sc_body)

    chunked = run(itemsT, banT)

    pi_t = pl.pallas_call(
        _tc_unchunk,
        grid=(_NW,),
        in_specs=[pl.BlockSpec((1, _ROWS, _NB), lambda i: (i, 0, 0))],
        out_specs=pl.BlockSpec((_S, _C, _NB), lambda i: (0, 0, i)),
        out_shape=jax.ShapeDtypeStruct((_S, _C, _N), jnp.float32),
        compiler_params=pltpu.CompilerParams(
            dimension_semantics=("parallel",),
        ),
    )(chunked)

    pi = jnp.transpose(pi_t, (2, 0, 1))
    v = jnp.zeros((_N, _S, 1), jnp.float32)
    return (pi, v)
